# Initial kernel scaffold; baseline (speedup 1.0000x reference)
import jax
import jax.numpy as jnp
from jax.experimental import pallas as pl


def kernel(x, edge_index, W1l, b1l, W1r, W2l, b2l, W2r, W3l, b3l, W3r, Wfc, bfc):
    raise NotImplementedError("write your pallas kernel here")


# speedup vs baseline: 12.7469x; 12.7469x over previous
import functools

import jax
import jax.numpy as jnp
from jax import lax
from jax.experimental import pallas as pl
from jax.experimental.pallas import tpu as pltpu
from jax.experimental.pallas import tpu_sc as plsc

N = 10000
E = 320000
D_IN = 128
H = 16
R = 237

LANES = 128
CHUNK_ROWS = 8
NCORES = 2
NSUB = 16
NTILES = NCORES * NSUB
E_PAD = 327680
ROWS2D = E_PAD // LANES
ROWS_PER_TILE = ROWS2D // NTILES
N_CHUNKS = ROWS_PER_TILE // CHUNK_ROWS
ACC_ROWS = 10240
ZROWS = ACC_ROWS // NSUB


def _make_seg_sum(with_deg):
    mesh = plsc.VectorSubcoreMesh(core_axis_name="c", subcore_axis_name="s")
    out_type = [jax.ShapeDtypeStruct((NCORES, ACC_ROWS, H), jnp.float32)]
    if with_deg:
        out_type.append(jax.ShapeDtypeStruct((NCORES, ACC_ROWS, H), jnp.float32))
    scratch_types = [
        pltpu.VMEM((CHUNK_ROWS, LANES), jnp.int32),
        pltpu.VMEM((CHUNK_ROWS, LANES), jnp.int32),
        pltpu.VMEM((CHUNK_ROWS, LANES, H), jnp.float32),
        pltpu.VMEM((ZROWS, H), jnp.float32),
        pltpu.VMEM((LANES, H), jnp.float32),
        pltpu.VMEM_SHARED((ACC_ROWS, H), jnp.float32),
        pltpu.VMEM_SHARED((ACC_ROWS, H), jnp.float32),
        pltpu.SemaphoreType.DMA,
    ]

    def body(p_hbm, src_hbm, dst_hbm, zeros_hbm, ones_hbm, *rest):
        if with_deg:
            acc_out, deg_out = rest[0], rest[1]
            scr = rest[2:]
        else:
            acc_out = rest[0]
            deg_out = None
            scr = rest[1:]
        src_v, dst_v, rows_v, stage_v, ones_v, acc_s, dacc_s, sem = scr
        cid = lax.axis_index("c")
        sid = lax.axis_index("s")

        pltpu.sync_copy(zeros_hbm, stage_v)
        pltpu.sync_copy(stage_v, acc_s.at[pl.ds(sid * ZROWS, ZROWS)])
        if with_deg:
            pltpu.sync_copy(stage_v, dacc_s.at[pl.ds(sid * ZROWS, ZROWS)])
            pltpu.sync_copy(ones_hbm, ones_v)
        plsc.subcore_barrier()

        wid = cid * NSUB + sid
        row0 = wid * ROWS_PER_TILE

        def chunk(g, carry):
            r0 = row0 + g * CHUNK_ROWS
            pltpu.sync_copy(src_hbm.at[pl.ds(r0, CHUNK_ROWS)], src_v)
            pltpu.sync_copy(dst_hbm.at[pl.ds(r0, CHUNK_ROWS)], dst_v)
            copies = [
                pltpu.async_copy(p_hbm.at[src_v.at[j]], rows_v.at[j], sem)
                for j in range(CHUNK_ROWS)
            ]
            for c in copies:
                c.wait()
            for j in range(CHUNK_ROWS):
                pltpu.sync_copy(rows_v.at[j], acc_s.at[dst_v.at[j]], add=True)
            if with_deg:
                for j in range(CHUNK_ROWS):
                    pltpu.sync_copy(ones_v, dacc_s.at[dst_v.at[j]], add=True)
            return carry

        lax.fori_loop(0, N_CHUNKS, chunk, 0)
        plsc.subcore_barrier()

        o0 = sid * ZROWS
        pltpu.sync_copy(acc_s.at[pl.ds(o0, ZROWS)], stage_v)
        pltpu.sync_copy(stage_v, acc_out.at[cid, pl.ds(o0, ZROWS)])
        if with_deg:
            pltpu.sync_copy(dacc_s.at[pl.ds(o0, ZROWS)], stage_v)
            pltpu.sync_copy(stage_v, deg_out.at[cid, pl.ds(o0, ZROWS)])

    return pl.kernel(body, out_type=out_type, mesh=mesh,
                     scratch_types=scratch_types,
                     compiler_params=pltpu.CompilerParams(
                         use_tc_tiling_on_sc=False))


_seg_sum_deg = _make_seg_sum(True)
_seg_sum = _make_seg_sum(False)


BN = 2000


def _dotT(a, w):
    return lax.dot_general(a, w, (((1,), (1,)), ((), ())),
                           preferred_element_type=jnp.float32)


def _stage1_body(x_ref, wl_ref, wr_ref, p_ref, q_ref):
    xb = x_ref[...]
    p_ref[...] = _dotT(xb, wl_ref[...])
    q_ref[...] = _dotT(xb, wr_ref[...])


def _tc_stage1(x, wl, wr):
    return pl.pallas_call(
        _stage1_body,
        grid=(N // BN,),
        in_specs=[
            pl.BlockSpec((BN, D_IN), lambda i: (i, 0)),
            pl.BlockSpec((H, D_IN), lambda i: (0, 0)),
            pl.BlockSpec((H, D_IN), lambda i: (0, 0)),
        ],
        out_specs=[pl.BlockSpec((BN, H), lambda i: (i, 0))] * 2,
        out_shape=[jax.ShapeDtypeStruct((N, H), jnp.float32)] * 2,
    )(x, wl, wr)


def _mid_first_body(a0_ref, a1_ref, d0_ref, d1_ref, q_ref, b_ref, wl_ref,
                    wr_ref, p2_ref, q2_ref, rdeg_ref):
    rdeg = 1.0 / jnp.maximum(d0_ref[...] + d1_ref[...], 1.0)
    h = jnp.maximum((a0_ref[...] + a1_ref[...]) * rdeg + b_ref[...] + q_ref[...], 0.0)
    p2_ref[...] = _dotT(h, wl_ref[...])
    q2_ref[...] = _dotT(h, wr_ref[...])
    rdeg_ref[...] = rdeg


def _tc_mid_first(a0, a1, d0, d1, q, b, wl, wr):
    nh = pl.BlockSpec((BN, H), lambda i: (i, 0))
    wspec = pl.BlockSpec((H, H), lambda i: (0, 0))
    return pl.pallas_call(
        _mid_first_body,
        grid=(N // BN,),
        in_specs=[nh, nh, nh, nh, nh, pl.BlockSpec((1, H), lambda i: (0, 0)),
                  wspec, wspec],
        out_specs=[nh, nh, nh],
        out_shape=[jax.ShapeDtypeStruct((N, H), jnp.float32)] * 3,
    )(a0, a1, d0, d1, q, b, wl, wr)


def _mid_body(a0_ref, a1_ref, rdeg_ref, q_ref, b_ref, wl_ref, wr_ref,
              p3_ref, q3_ref):
    h = jnp.maximum((a0_ref[...] + a1_ref[...]) * rdeg_ref[...]
                    + b_ref[...] + q_ref[...], 0.0)
    p3_ref[...] = _dotT(h, wl_ref[...])
    q3_ref[...] = _dotT(h, wr_ref[...])


def _tc_mid(a0, a1, rdeg, q, b, wl, wr):
    nh = pl.BlockSpec((BN, H), lambda i: (i, 0))
    wspec = pl.BlockSpec((H, H), lambda i: (0, 0))
    return pl.pallas_call(
        _mid_body,
        grid=(N // BN,),
        in_specs=[nh, nh, nh, nh, pl.BlockSpec((1, H), lambda i: (0, 0)),
                  wspec, wspec],
        out_specs=[nh, nh],
        out_shape=[jax.ShapeDtypeStruct((N, H), jnp.float32)] * 2,
    )(a0, a1, rdeg, q, b, wl, wr)


def _tail_body(a0_ref, a1_ref, rdeg_ref, q_ref, b_ref, wfc_ref, bfc_ref,
               out_ref):
    h = (a0_ref[...] + a1_ref[...]) * rdeg_ref[...] + b_ref[...] + q_ref[...]
    out_ref[...] = _dotT(h, wfc_ref[...]) + bfc_ref[...]


def _tc_tail(a0, a1, rdeg, q, b, wfc, bfc):
    nh = pl.BlockSpec((BN, H), lambda i: (i, 0))
    return pl.pallas_call(
        _tail_body,
        grid=(N // BN,),
        in_specs=[nh, nh, nh, nh, pl.BlockSpec((1, H), lambda i: (0, 0)),
                  pl.BlockSpec((R, H), lambda i: (0, 0)),
                  pl.BlockSpec((1, R), lambda i: (0, 0))],
        out_specs=pl.BlockSpec((BN, R), lambda i: (i, 0)),
        out_shape=jax.ShapeDtypeStruct((N, R), jnp.float32),
    )(a0, a1, rdeg, q, b, wfc, bfc)


@jax.jit
def kernel(x, edge_index, W1l, b1l, W1r, W2l, b2l, W2r, W3l, b3l, W3r, Wfc, bfc):
    pad = E_PAD - E
    src2d = jnp.concatenate(
        [edge_index[0], jnp.zeros((pad,), jnp.int32)]).reshape(ROWS2D, LANES)
    dst2d = jnp.concatenate(
        [edge_index[1], jnp.full((pad,), N, jnp.int32)]).reshape(ROWS2D, LANES)
    zeros_c = jnp.zeros((ZROWS, H), jnp.float32)
    ones_c = jnp.ones((LANES, H), jnp.float32)

    p1, q1 = _tc_stage1(x, W1l, W1r)
    acc1, deg = _seg_sum_deg(p1, src2d, dst2d, zeros_c, ones_c)
    p2, q2, rdeg = _tc_mid_first(acc1[0, :N], acc1[1, :N], deg[0, :N],
                                 deg[1, :N], q1, b1l.reshape(1, H), W2l, W2r)
    acc2 = _seg_sum(p2, src2d, dst2d, zeros_c, ones_c)[0]
    p3, q3 = _tc_mid(acc2[0, :N], acc2[1, :N], rdeg, q2, b2l.reshape(1, H),
                     W3l, W3r)
    acc3 = _seg_sum(p3, src2d, dst2d, zeros_c, ones_c)[0]
    return _tc_tail(acc3[0, :N], acc3[1, :N], rdeg, q3, b3l.reshape(1, H),
                    Wfc, bfc.reshape(1, R))


# speedup vs baseline: 14.1514x; 1.1102x over previous
import functools

import jax
import jax.numpy as jnp
from jax import lax
from jax.experimental import pallas as pl
from jax.experimental.pallas import tpu as pltpu
from jax.experimental.pallas import tpu_sc as plsc

N = 10000
E = 320000
D_IN = 128
H = 16
R = 237

LANES = 128
CHUNK_ROWS = 8
NCORES = 2
NSUB = 16
NTILES = NCORES * NSUB
E_PAD = 327680
ROWS2D = E_PAD // LANES
ROWS_PER_TILE = ROWS2D // NTILES
N_CHUNKS = ROWS_PER_TILE // CHUNK_ROWS
ACC_ROWS = 10240
ZROWS = ACC_ROWS // NSUB


def _make_seg_sum(with_deg):
    mesh = plsc.VectorSubcoreMesh(core_axis_name="c", subcore_axis_name="s")
    out_type = [jax.ShapeDtypeStruct((NCORES, ACC_ROWS, H), jnp.float32)]
    if with_deg:
        out_type.append(jax.ShapeDtypeStruct((NCORES, ACC_ROWS, H), jnp.float32))
    scratch_types = [
        pltpu.VMEM((2, CHUNK_ROWS, LANES), jnp.int32),
        pltpu.VMEM((2, CHUNK_ROWS, LANES), jnp.int32),
        pltpu.VMEM((2, CHUNK_ROWS, LANES, H), jnp.float32),
        pltpu.VMEM((LANES, H), jnp.float32),
        pltpu.VMEM_SHARED((ACC_ROWS, H), jnp.float32),
        pltpu.VMEM_SHARED((ACC_ROWS, H), jnp.float32),
        pltpu.SemaphoreType.DMA,
        pltpu.SemaphoreType.DMA,
    ]

    def body(p_hbm, src_hbm, dst_hbm, zeros_hbm, ones_hbm, *rest):
        if with_deg:
            acc_out, deg_out = rest[0], rest[1]
            scr = rest[2:]
        else:
            acc_out = rest[0]
            deg_out = None
            scr = rest[1:]
        src_v, dst_v, rows_v, ones_v, acc_s, dacc_s, sem_g, sem_s = scr
        cid = lax.axis_index("c")
        sid = lax.axis_index("s")
        wid = cid * NSUB + sid
        row0 = wid * ROWS_PER_TILE

        def load_and_fire(slot, g):
            r0 = row0 + g * CHUNK_ROWS
            pltpu.sync_copy(src_hbm.at[pl.ds(r0, CHUNK_ROWS)], src_v.at[slot])
            pltpu.sync_copy(dst_hbm.at[pl.ds(r0, CHUNK_ROWS)], dst_v.at[slot])
            for j in range(CHUNK_ROWS):
                pltpu.async_copy(p_hbm.at[src_v.at[slot, j]],
                                 rows_v.at[slot, j], sem_g)

        def drain_gathers(slot):
            for j in range(CHUNK_ROWS):
                pltpu.make_async_copy(p_hbm.at[src_v.at[slot, j]],
                                      rows_v.at[slot, j], sem_g).wait()

        def fire_scatters(slot):
            for j in range(CHUNK_ROWS):
                pltpu.async_copy(rows_v.at[slot, j],
                                 acc_s.at[dst_v.at[slot, j]], sem_s, add=True)
            if with_deg:
                for j in range(CHUNK_ROWS):
                    pltpu.async_copy(ones_v, dacc_s.at[dst_v.at[slot, j]],
                                     sem_s, add=True)

        def drain_scatters(slot):
            for j in range(CHUNK_ROWS):
                pltpu.make_async_copy(rows_v.at[slot, j],
                                      acc_s.at[dst_v.at[slot, j]], sem_s).wait()
            if with_deg:
                for j in range(CHUNK_ROWS):
                    pltpu.make_async_copy(ones_v, dacc_s.at[dst_v.at[slot, j]],
                                          sem_s).wait()

        load_and_fire(0, 0)
        if with_deg:
            pltpu.sync_copy(ones_hbm, ones_v)
        z0 = sid * ZROWS
        pltpu.sync_copy(zeros_hbm, acc_s.at[pl.ds(z0, ZROWS)])
        if with_deg:
            pltpu.sync_copy(zeros_hbm, dacc_s.at[pl.ds(z0, ZROWS)])
        plsc.subcore_barrier()

        def step(g, carry):
            b = lax.rem(g, 2)
            nb = 1 - b
            drain_gathers(b)

            @pl.when(g > 0)
            def _():
                drain_scatters(nb)

            @pl.when(g < N_CHUNKS - 1)
            def _():
                load_and_fire(nb, g + 1)

            fire_scatters(b)
            return carry

        lax.fori_loop(0, N_CHUNKS, step, 0)
        drain_scatters((N_CHUNKS - 1) % 2)
        plsc.subcore_barrier()

        pltpu.sync_copy(acc_s.at[pl.ds(z0, ZROWS)],
                        acc_out.at[cid, pl.ds(z0, ZROWS)])
        if with_deg:
            pltpu.sync_copy(dacc_s.at[pl.ds(z0, ZROWS)],
                            deg_out.at[cid, pl.ds(z0, ZROWS)])

    return pl.kernel(body, out_type=out_type, mesh=mesh,
                     scratch_types=scratch_types,
                     compiler_params=pltpu.CompilerParams(
                         use_tc_tiling_on_sc=False))


_seg_sum_deg = _make_seg_sum(True)
_seg_sum = _make_seg_sum(False)


BN = 2000


def _dotT(a, w):
    return lax.dot_general(a, w, (((1,), (1,)), ((), ())),
                           preferred_element_type=jnp.float32)


def _stage1_body(x_ref, wl_ref, wr_ref, p_ref, q_ref):
    xb = x_ref[...]
    p_ref[...] = _dotT(xb, wl_ref[...])
    q_ref[...] = _dotT(xb, wr_ref[...])


def _tc_stage1(x, wl, wr):
    return pl.pallas_call(
        _stage1_body,
        grid=(N // BN,),
        in_specs=[
            pl.BlockSpec((BN, D_IN), lambda i: (i, 0)),
            pl.BlockSpec((H, D_IN), lambda i: (0, 0)),
            pl.BlockSpec((H, D_IN), lambda i: (0, 0)),
        ],
        out_specs=[pl.BlockSpec((BN, H), lambda i: (i, 0))] * 2,
        out_shape=[jax.ShapeDtypeStruct((N, H), jnp.float32)] * 2,
    )(x, wl, wr)


def _mid_first_body(a0_ref, a1_ref, d0_ref, d1_ref, q_ref, b_ref, wl_ref,
                    wr_ref, p2_ref, q2_ref, rdeg_ref):
    rdeg = 1.0 / jnp.maximum(d0_ref[...] + d1_ref[...], 1.0)
    h = jnp.maximum((a0_ref[...] + a1_ref[...]) * rdeg + b_ref[...] + q_ref[...], 0.0)
    p2_ref[...] = _dotT(h, wl_ref[...])
    q2_ref[...] = _dotT(h, wr_ref[...])
    rdeg_ref[...] = rdeg


def _tc_mid_first(a0, a1, d0, d1, q, b, wl, wr):
    nh = pl.BlockSpec((BN, H), lambda i: (i, 0))
    wspec = pl.BlockSpec((H, H), lambda i: (0, 0))
    return pl.pallas_call(
        _mid_first_body,
        grid=(N // BN,),
        in_specs=[nh, nh, nh, nh, nh, pl.BlockSpec((1, H), lambda i: (0, 0)),
                  wspec, wspec],
        out_specs=[nh, nh, nh],
        out_shape=[jax.ShapeDtypeStruct((N, H), jnp.float32)] * 3,
    )(a0, a1, d0, d1, q, b, wl, wr)


def _mid_body(a0_ref, a1_ref, rdeg_ref, q_ref, b_ref, wl_ref, wr_ref,
              p3_ref, q3_ref):
    h = jnp.maximum((a0_ref[...] + a1_ref[...]) * rdeg_ref[...]
                    + b_ref[...] + q_ref[...], 0.0)
    p3_ref[...] = _dotT(h, wl_ref[...])
    q3_ref[...] = _dotT(h, wr_ref[...])


def _tc_mid(a0, a1, rdeg, q, b, wl, wr):
    nh = pl.BlockSpec((BN, H), lambda i: (i, 0))
    wspec = pl.BlockSpec((H, H), lambda i: (0, 0))
    return pl.pallas_call(
        _mid_body,
        grid=(N // BN,),
        in_specs=[nh, nh, nh, nh, pl.BlockSpec((1, H), lambda i: (0, 0)),
                  wspec, wspec],
        out_specs=[nh, nh],
        out_shape=[jax.ShapeDtypeStruct((N, H), jnp.float32)] * 2,
    )(a0, a1, rdeg, q, b, wl, wr)


def _tail_body(a0_ref, a1_ref, rdeg_ref, q_ref, b_ref, wfc_ref, bfc_ref,
               out_ref):
    h = (a0_ref[...] + a1_ref[...]) * rdeg_ref[...] + b_ref[...] + q_ref[...]
    out_ref[...] = _dotT(h, wfc_ref[...]) + bfc_ref[...]


def _tc_tail(a0, a1, rdeg, q, b, wfc, bfc):
    nh = pl.BlockSpec((BN, H), lambda i: (i, 0))
    return pl.pallas_call(
        _tail_body,
        grid=(N // BN,),
        in_specs=[nh, nh, nh, nh, pl.BlockSpec((1, H), lambda i: (0, 0)),
                  pl.BlockSpec((R, H), lambda i: (0, 0)),
                  pl.BlockSpec((1, R), lambda i: (0, 0))],
        out_specs=pl.BlockSpec((BN, R), lambda i: (i, 0)),
        out_shape=jax.ShapeDtypeStruct((N, R), jnp.float32),
    )(a0, a1, rdeg, q, b, wfc, bfc)


@jax.jit
def kernel(x, edge_index, W1l, b1l, W1r, W2l, b2l, W2r, W3l, b3l, W3r, Wfc, bfc):
    pad = E_PAD - E
    src2d = jnp.concatenate(
        [edge_index[0], jnp.zeros((pad,), jnp.int32)]).reshape(ROWS2D, LANES)
    dst2d = jnp.concatenate(
        [edge_index[1], jnp.full((pad,), N, jnp.int32)]).reshape(ROWS2D, LANES)
    zeros_c = jnp.zeros((ZROWS, H), jnp.float32)
    ones_c = jnp.ones((LANES, H), jnp.float32)

    p1, q1 = _tc_stage1(x, W1l, W1r)
    acc1, deg = _seg_sum_deg(p1, src2d, dst2d, zeros_c, ones_c)
    p2, q2, rdeg = _tc_mid_first(acc1[0, :N], acc1[1, :N], deg[0, :N],
                                 deg[1, :N], q1, b1l.reshape(1, H), W2l, W2r)
    acc2 = _seg_sum(p2, src2d, dst2d, zeros_c, ones_c)[0]
    p3, q3 = _tc_mid(acc2[0, :N], acc2[1, :N], rdeg, q2, b2l.reshape(1, H),
                     W3l, W3r)
    acc3 = _seg_sum(p3, src2d, dst2d, zeros_c, ones_c)[0]
    return _tc_tail(acc3[0, :N], acc3[1, :N], rdeg, q3, b3l.reshape(1, H),
                    Wfc, bfc.reshape(1, R))
